# baseline (speedup 1.0000x reference)
import jax
import jax.numpy as jnp
import numpy as np
from jax.experimental import pallas as pl
from jax.experimental.pallas import tpu as pltpu

_N = 50000
_C = 1000
_N_BINS = 15
_BLK_ROWS = 2000
_GRID = _N // _BLK_ROWS


def _make_rank_plan():
    q = np.asarray(jnp.linspace(0.0, float(_N), _N_BINS + 1), np.float32)
    lo_ranks = []
    fracs = []
    for k in range(1, _N_BINS):
        i = int(np.floor(q[k]))
        i = min(i, _N - 2)
        lo_ranks.append(i)
        fracs.append(np.float32(q[k] - np.float32(i)))
    return np.array(lo_ranks, np.int32), np.array(fracs, np.float32)


_LO_RANKS, _FRACS = _make_rank_plan()


def _conf_acc_body(x_ref, lab_ref, conf_ref, acc_ref):
    x = x_ref[...]
    m = jnp.max(x, axis=1)
    iota = jax.lax.broadcasted_iota(jnp.int32, (_BLK_ROWS, _C), 1)
    amax = jnp.min(jnp.where(x == m[:, None], iota, _C), axis=1)
    s = jnp.sum(jnp.exp(x - m[:, None]), axis=1)
    conf_ref[0, 0, :] = 1.0 / s
    acc_ref[0, 0, :] = (amax == lab_ref[0, 0, :]).astype(jnp.float32)


def _conf_acc(logits, labels):
    lab3 = labels.reshape(_GRID, 1, _BLK_ROWS)
    return pl.pallas_call(
        _conf_acc_body,
        grid=(_GRID,),
        in_specs=[
            pl.BlockSpec((_BLK_ROWS, _C), lambda i: (i, 0)),
            pl.BlockSpec((1, 1, _BLK_ROWS), lambda i: (i, 0, 0)),
        ],
        out_specs=[
            pl.BlockSpec((1, 1, _BLK_ROWS), lambda i: (i, 0, 0)),
            pl.BlockSpec((1, 1, _BLK_ROWS), lambda i: (i, 0, 0)),
        ],
        out_shape=[
            jax.ShapeDtypeStruct((_GRID, 1, _BLK_ROWS), jnp.float32),
            jax.ShapeDtypeStruct((_GRID, 1, _BLK_ROWS), jnp.float32),
        ],
        compiler_params=pltpu.CompilerParams(
            dimension_semantics=("arbitrary",)
        ),
    )(logits, lab3)


def _ece_body(conf_ref, acc_ref, ranks_ref, fr_ref, out_ref):
    conf = conf_ref[:, 0, :]
    acc = acc_ref[:, 0, :]
    bits = jax.lax.bitcast_convert_type(conf, jnp.int32)

    ranks = ranks_ref[...]
    nt = _N_BINS - 1

    def step(i, cur):
        b = 30 - i
        test = cur + (jnp.left_shift(jnp.int32(1), b) - 1)
        cnt = jnp.sum(
            (bits[None, :, :] <= test[:, None, None]).astype(jnp.int32),
            axis=(1, 2),
        )
        keep = cnt >= ranks + 1
        return jnp.where(keep, cur, cur + jnp.left_shift(jnp.int32(1), b))

    lo_bits = jax.lax.fori_loop(0, 31, step, jnp.zeros((nt,), jnp.int32))
    lo_vals = jax.lax.bitcast_convert_type(lo_bits, jnp.float32)

    cnt_le = jnp.sum(
        (bits[None, :, :] <= lo_bits[:, None, None]).astype(jnp.int32),
        axis=(1, 2),
    )
    nxt = jnp.min(
        jnp.where(conf[None, :, :] > lo_vals[:, None, None],
                  conf[None, :, :], jnp.float32(2.0)),
        axis=(1, 2),
    )
    hi_vals = jnp.where(cnt_le >= ranks + 2, lo_vals, nxt)

    mids = lo_vals + fr_ref[...] * (hi_vals - lo_vals)

    b0 = jnp.min(conf)
    b15 = jnp.max(conf)
    bl = [b0] + [mids[t] for t in range(nt)] + [b15]

    inv_n = jnp.float32(1.0 / _N)
    ece = jnp.float32(0.0)
    for k in range(_N_BINS):
        lo = bl[k]
        hi = bl[k + 1]
        mf = ((conf > lo) & (conf <= hi)).astype(jnp.float32)
        cnt = jnp.sum(mf)
        sc = jnp.sum(conf * mf)
        sa = jnp.sum(acc * mf)
        denom = jnp.where(cnt > 0, cnt, jnp.float32(1.0))
        contrib = jnp.abs(sc / denom - sa / denom) * (cnt * inv_n)
        ece = ece + jnp.where(cnt > 0, contrib, jnp.float32(0.0))
    out_ref[...] = jnp.full((8, 128), ece, jnp.float32)


def _ece(conf2, acc2):
    out = pl.pallas_call(
        _ece_body,
        grid=(1,),
        in_specs=[
            pl.BlockSpec((_GRID, 1, _BLK_ROWS), lambda i: (0, 0, 0)),
            pl.BlockSpec((_GRID, 1, _BLK_ROWS), lambda i: (0, 0, 0)),
            pl.BlockSpec((_N_BINS - 1,), lambda i: (0,)),
            pl.BlockSpec((_N_BINS - 1,), lambda i: (0,)),
        ],
        out_specs=pl.BlockSpec((8, 128), lambda i: (0, 0)),
        out_shape=jax.ShapeDtypeStruct((8, 128), jnp.float32),
    )(conf2, acc2, jnp.asarray(_LO_RANKS), jnp.asarray(_FRACS))
    return out[0, 0:1]


@jax.jit
def kernel(logits, labels):
    conf2, acc2 = _conf_acc(logits, labels)
    return _ece(conf2, acc2)


# speedup vs baseline: 1.0379x; 1.0379x over previous
import jax
import jax.numpy as jnp
import numpy as np
from jax.experimental import pallas as pl
from jax.experimental.pallas import tpu as pltpu

_N = 50000
_C = 1000
_N_BINS = 15
_BLK_ROWS = 2000
_GRID = _N // _BLK_ROWS


def _make_rank_plan():
    q = np.asarray(jnp.linspace(0.0, float(_N), _N_BINS + 1), np.float32)
    lo_ranks = []
    fracs = []
    for k in range(1, _N_BINS):
        i = int(np.floor(q[k]))
        i = min(i, _N - 2)
        lo_ranks.append(i)
        fracs.append(np.float32(q[k] - np.float32(i)))
    return np.array(lo_ranks, np.int32), np.array(fracs, np.float32)


_LO_RANKS, _FRACS = _make_rank_plan()


def _conf_acc_body(x_ref, lab_ref, conf_ref, acc_ref):
    x = x_ref[...]
    m = jnp.max(x, axis=1)
    iota = jax.lax.broadcasted_iota(jnp.int32, (_BLK_ROWS, _C), 1)
    amax = jnp.min(jnp.where(x == m[:, None], iota, _C), axis=1)
    s = jnp.sum(jnp.exp(x - m[:, None]), axis=1)
    conf_ref[0, 0, :] = 1.0 / s
    acc_ref[0, 0, :] = (amax == lab_ref[0, 0, :]).astype(jnp.float32)


def _conf_acc(logits, labels):
    lab3 = labels.reshape(_GRID, 1, _BLK_ROWS)
    return pl.pallas_call(
        _conf_acc_body,
        grid=(_GRID,),
        in_specs=[
            pl.BlockSpec((_BLK_ROWS, _C), lambda i: (i, 0)),
            pl.BlockSpec((1, 1, _BLK_ROWS), lambda i: (i, 0, 0)),
        ],
        out_specs=[
            pl.BlockSpec((1, 1, _BLK_ROWS), lambda i: (i, 0, 0)),
            pl.BlockSpec((1, 1, _BLK_ROWS), lambda i: (i, 0, 0)),
        ],
        out_shape=[
            jax.ShapeDtypeStruct((_GRID, 1, _BLK_ROWS), jnp.float32),
            jax.ShapeDtypeStruct((_GRID, 1, _BLK_ROWS), jnp.float32),
        ],
        compiler_params=pltpu.CompilerParams(
            dimension_semantics=("parallel",)
        ),
    )(logits, lab3)


def _ece_body(conf_ref, acc_ref, ranks_ref, fr_ref, out_ref):
    conf = conf_ref[:, 0, :]
    acc = acc_ref[:, 0, :]
    bits = jax.lax.bitcast_convert_type(conf, jnp.int32)

    ranks = ranks_ref[...]
    nt = _N_BINS - 1

    def step(i, cur):
        b = 26 - i
        test = cur + (jnp.left_shift(jnp.int32(1), b) - 1)
        cnt = jnp.sum(
            (bits[None, :, :] <= test[:, None, None]).astype(jnp.int32),
            axis=(1, 2),
        )
        keep = cnt >= ranks + 1
        return jnp.where(keep, cur, cur + jnp.left_shift(jnp.int32(1), b))

    lo_bits = jax.lax.fori_loop(
        0, 27, step, jnp.full((nt,), 0x38000000, jnp.int32)
    )
    lo_vals = jax.lax.bitcast_convert_type(lo_bits, jnp.float32)

    cnt_le = jnp.sum(
        (bits[None, :, :] <= lo_bits[:, None, None]).astype(jnp.int32),
        axis=(1, 2),
    )
    nxt = jnp.min(
        jnp.where(conf[None, :, :] > lo_vals[:, None, None],
                  conf[None, :, :], jnp.float32(2.0)),
        axis=(1, 2),
    )
    hi_vals = jnp.where(cnt_le >= ranks + 2, lo_vals, nxt)

    mids = lo_vals + fr_ref[...] * (hi_vals - lo_vals)

    b0 = jnp.min(conf)
    b15 = jnp.max(conf)
    bl = [b0] + [mids[t] for t in range(nt)] + [b15]

    inv_n = jnp.float32(1.0 / _N)
    ece = jnp.float32(0.0)
    for k in range(_N_BINS):
        lo = bl[k]
        hi = bl[k + 1]
        mf = ((conf > lo) & (conf <= hi)).astype(jnp.float32)
        cnt = jnp.sum(mf)
        sc = jnp.sum(conf * mf)
        sa = jnp.sum(acc * mf)
        denom = jnp.where(cnt > 0, cnt, jnp.float32(1.0))
        contrib = jnp.abs(sc / denom - sa / denom) * (cnt * inv_n)
        ece = ece + jnp.where(cnt > 0, contrib, jnp.float32(0.0))
    out_ref[...] = jnp.full((8, 128), ece, jnp.float32)


def _ece(conf2, acc2):
    out = pl.pallas_call(
        _ece_body,
        grid=(1,),
        in_specs=[
            pl.BlockSpec((_GRID, 1, _BLK_ROWS), lambda i: (0, 0, 0)),
            pl.BlockSpec((_GRID, 1, _BLK_ROWS), lambda i: (0, 0, 0)),
            pl.BlockSpec((_N_BINS - 1,), lambda i: (0,)),
            pl.BlockSpec((_N_BINS - 1,), lambda i: (0,)),
        ],
        out_specs=pl.BlockSpec((8, 128), lambda i: (0, 0)),
        out_shape=jax.ShapeDtypeStruct((8, 128), jnp.float32),
    )(conf2, acc2, jnp.asarray(_LO_RANKS), jnp.asarray(_FRACS))
    return out[0, 0:1]


@jax.jit
def kernel(logits, labels):
    conf2, acc2 = _conf_acc(logits, labels)
    return _ece(conf2, acc2)


# speedup vs baseline: 1.0460x; 1.0078x over previous
import jax
import jax.numpy as jnp
import numpy as np
from jax.experimental import pallas as pl
from jax.experimental.pallas import tpu as pltpu

_N = 50000
_C = 1000
_N_BINS = 15
_BLK_ROWS = 2000
_GRID = _N // _BLK_ROWS


def _make_rank_plan():
    q = np.asarray(jnp.linspace(0.0, float(_N), _N_BINS + 1), np.float32)
    lo_ranks = []
    fracs = []
    for k in range(1, _N_BINS):
        i = int(np.floor(q[k]))
        i = min(i, _N - 2)
        lo_ranks.append(i)
        fracs.append(np.float32(q[k] - np.float32(i)))
    return np.array(lo_ranks, np.int32), np.array(fracs, np.float32)


_LO_RANKS, _FRACS = _make_rank_plan()


def _conf_acc_body(x_ref, lab_ref, conf_ref, acc_ref):
    x = x_ref[...]
    m = jnp.max(x, axis=1)
    iota = jax.lax.broadcasted_iota(jnp.int32, (_BLK_ROWS, _C), 1)
    amax = jnp.min(jnp.where(x == m[:, None], iota, _C), axis=1)
    e = jnp.exp(x - m[:, None])
    ones = jnp.ones((_C, 1), jnp.float32)
    s = jax.lax.dot_general(
        e, ones, (((1,), (0,)), ((), ())),
        preferred_element_type=jnp.float32,
    )[:, 0]
    conf_ref[0, 0, :] = 1.0 / s
    acc_ref[0, 0, :] = (amax == lab_ref[0, 0, :]).astype(jnp.float32)


def _conf_acc(logits, labels):
    lab3 = labels.reshape(_GRID, 1, _BLK_ROWS)
    return pl.pallas_call(
        _conf_acc_body,
        grid=(_GRID,),
        in_specs=[
            pl.BlockSpec((_BLK_ROWS, _C), lambda i: (i, 0)),
            pl.BlockSpec((1, 1, _BLK_ROWS), lambda i: (i, 0, 0)),
        ],
        out_specs=[
            pl.BlockSpec((1, 1, _BLK_ROWS), lambda i: (i, 0, 0)),
            pl.BlockSpec((1, 1, _BLK_ROWS), lambda i: (i, 0, 0)),
        ],
        out_shape=[
            jax.ShapeDtypeStruct((_GRID, 1, _BLK_ROWS), jnp.float32),
            jax.ShapeDtypeStruct((_GRID, 1, _BLK_ROWS), jnp.float32),
        ],
        compiler_params=pltpu.CompilerParams(
            dimension_semantics=("parallel",)
        ),
    )(logits, lab3)


def _ece_body(conf_ref, acc_ref, ranks_ref, fr_ref, out_ref):
    conf = conf_ref[:, 0, :]
    acc = acc_ref[:, 0, :]
    bits = jax.lax.bitcast_convert_type(conf, jnp.int32)

    ranks = ranks_ref[...]
    nt = _N_BINS - 1

    def step(i, cur):
        b = 26 - i
        test = cur + (jnp.left_shift(jnp.int32(1), b) - 1)
        cnt = jnp.sum(
            (bits[None, :, :] <= test[:, None, None]).astype(jnp.int32),
            axis=(1, 2),
        )
        keep = cnt >= ranks + 1
        return jnp.where(keep, cur, cur + jnp.left_shift(jnp.int32(1), b))

    lo_bits = jax.lax.fori_loop(
        0, 27, step, jnp.full((nt,), 0x38000000, jnp.int32)
    )
    lo_vals = jax.lax.bitcast_convert_type(lo_bits, jnp.float32)

    cnt_le = jnp.sum(
        (bits[None, :, :] <= lo_bits[:, None, None]).astype(jnp.int32),
        axis=(1, 2),
    )
    nxt = jnp.min(
        jnp.where(conf[None, :, :] > lo_vals[:, None, None],
                  conf[None, :, :], jnp.float32(2.0)),
        axis=(1, 2),
    )
    hi_vals = jnp.where(cnt_le >= ranks + 2, lo_vals, nxt)

    mids = lo_vals + fr_ref[...] * (hi_vals - lo_vals)

    b0 = jnp.min(conf)
    b15 = jnp.max(conf)
    bl = [b0] + [mids[t] for t in range(nt)] + [b15]

    inv_n = jnp.float32(1.0 / _N)
    ece = jnp.float32(0.0)
    for k in range(_N_BINS):
        lo = bl[k]
        hi = bl[k + 1]
        mf = ((conf > lo) & (conf <= hi)).astype(jnp.float32)
        cnt = jnp.sum(mf)
        sc = jnp.sum(conf * mf)
        sa = jnp.sum(acc * mf)
        denom = jnp.where(cnt > 0, cnt, jnp.float32(1.0))
        contrib = jnp.abs(sc / denom - sa / denom) * (cnt * inv_n)
        ece = ece + jnp.where(cnt > 0, contrib, jnp.float32(0.0))
    out_ref[...] = jnp.full((8, 128), ece, jnp.float32)


def _ece(conf2, acc2):
    out = pl.pallas_call(
        _ece_body,
        grid=(1,),
        in_specs=[
            pl.BlockSpec((_GRID, 1, _BLK_ROWS), lambda i: (0, 0, 0)),
            pl.BlockSpec((_GRID, 1, _BLK_ROWS), lambda i: (0, 0, 0)),
            pl.BlockSpec((_N_BINS - 1,), lambda i: (0,)),
            pl.BlockSpec((_N_BINS - 1,), lambda i: (0,)),
        ],
        out_specs=pl.BlockSpec((8, 128), lambda i: (0, 0)),
        out_shape=jax.ShapeDtypeStruct((8, 128), jnp.float32),
    )(conf2, acc2, jnp.asarray(_LO_RANKS), jnp.asarray(_FRACS))
    return out[0, 0:1]


@jax.jit
def kernel(logits, labels):
    conf2, acc2 = _conf_acc(logits, labels)
    return _ece(conf2, acc2)
